# baseline (speedup 1.0000x reference)
import jax
import jax.numpy as jnp
from jax import lax
from jax.experimental import pallas as pl
from jax.experimental.pallas import tpu as pltpu
from jax.experimental.pallas import tpu_sc as plsc

EMBED_DIM = 32
SEQ = 200
NUM_CORES = 2
NUM_SUBCORES = 16
NUM_WORKERS = NUM_CORES * NUM_SUBCORES
NROWS = 4096
BPW = 128
TG = 5
NGT = SEQ // TG
LANE_PAD = 129


def _emb_body(table_hbm, idxt_hbm, out_hbm,
              idx_v, gbuf0, gbuf1, tbuf0, tbuf1,
              isem, gsem0, gsem1, osem0, osem1):
    bt = lax.axis_index("s") * NUM_CORES + lax.axis_index("c")
    gbuf = (gbuf0, gbuf1)
    tbuf = (tbuf0, tbuf1)
    gsem = (gsem0, gsem1)
    osem = (osem0, osem1)

    pltpu.async_copy(idxt_hbm.at[:, bt], idx_v, isem).wait()

    viota = lax.broadcasted_iota(jnp.int32, (16,), 0)
    esv = viota % 8
    etv0 = viota // 8
    etv1 = etv0 + 2

    def fire(g, b):
        for r in range(TG):
            pltpu.async_copy(
                table_hbm.at[idx_v.at[g * TG + r]],
                gbuf[b].at[r],
                gsem[b],
            )

    def drain_gathers(b):
        for r in range(TG):
            pltpu.make_async_copy(
                table_hbm.at[pl.ds(0, BPW)], gbuf[b].at[r], gsem[b]
            ).wait()

    def transpose(b):
        gb, tb = gbuf[b], tbuf[b]
        for tq in range(TG):
            tv = jnp.full((16,), tq, jnp.int32)

            @plsc.parallel_loop(0, BPW, unroll=2)
            def _tr_body(bl):
                blv = jnp.full((16,), 0, jnp.int32) + bl
                v0 = gb[tq, bl, pl.ds(0, 16)]
                v1 = gb[tq, bl, pl.ds(16, 16)]
                plsc.store_scatter(tb, [tv, etv0, esv, blv], v0)
                plsc.store_scatter(tb, [tv, etv1, esv, blv], v1)

    def out_start(g, b):
        pltpu.async_copy(
            tbuf[b].at[:, :, :, pl.ds(0, BPW)],
            out_hbm.at[pl.ds(g * TG, TG), :, bt],
            osem[b],
        )

    def out_wait(b):
        pltpu.make_async_copy(
            tbuf[b].at[:, :, :, pl.ds(0, BPW)],
            out_hbm.at[pl.ds(0, TG), :, bt],
            osem[b],
        ).wait()

    fire(0, 0)

    def phase(g, b, first, last):
        drain_gathers(b)
        if not last:
            fire(g + 1, 1 - b)
        if not first:
            out_wait(b)
        transpose(b)
        out_start(g, b)

    phase(0, 0, first=True, last=False)
    phase(1, 1, first=True, last=False)

    def steady(i, carry):
        g = 2 * i
        phase(g, 0, first=False, last=False)
        phase(g + 1, 1, first=False, last=False)
        return carry

    lax.fori_loop(1, NGT // 2 - 1, steady, 0)

    phase(NGT - 2, 0, first=False, last=False)
    phase(NGT - 1, 1, first=False, last=True)
    out_wait(0)
    out_wait(1)


_emb = pl.kernel(
    _emb_body,
    out_type=jax.ShapeDtypeStruct((SEQ, 4, NUM_WORKERS, 8, BPW), jnp.float32),
    mesh=plsc.VectorSubcoreMesh(core_axis_name="c", subcore_axis_name="s"),
    scratch_types=[
        pltpu.VMEM((SEQ, BPW), jnp.int32),
        pltpu.VMEM((TG, BPW, EMBED_DIM), jnp.float32),
        pltpu.VMEM((TG, BPW, EMBED_DIM), jnp.float32),
        pltpu.VMEM((TG, 4, 8, LANE_PAD), jnp.float32),
        pltpu.VMEM((TG, 4, 8, LANE_PAD), jnp.float32),
        pltpu.SemaphoreType.DMA,
        pltpu.SemaphoreType.DMA,
        pltpu.SemaphoreType.DMA,
        pltpu.SemaphoreType.DMA,
        pltpu.SemaphoreType.DMA,
    ],
    compiler_params=pltpu.CompilerParams(
        use_tc_tiling_on_sc=False, needs_layout_passes=False),
)


@jax.jit
def kernel(x, table):
    xt = jnp.transpose(x.astype(jnp.int32)).reshape(SEQ, NUM_WORKERS, BPW)
    out5 = _emb(table, xt)
    return jnp.transpose(out5, (2, 4, 0, 1, 3)).reshape(NROWS, SEQ, EMBED_DIM)


# speedup vs baseline: 1.0057x; 1.0057x over previous
import jax
import jax.numpy as jnp
from jax import lax
from jax.experimental import pallas as pl
from jax.experimental.pallas import tpu as pltpu
from jax.experimental.pallas import tpu_sc as plsc

EMBED_DIM = 32
SEQ = 200
NUM_CORES = 2
NUM_SUBCORES = 16
NUM_WORKERS = NUM_CORES * NUM_SUBCORES
NROWS = 4096
BPW = 128
TG = 5
NGT = SEQ // TG
LANE_PAD = 129


def _emb_body(table_hbm, idxt_hbm, out_hbm,
              idx_v, gbuf0, gbuf1, tbuf0, tbuf1,
              isem, gsem0, gsem1, osem0, osem1):
    bt = lax.axis_index("s") * NUM_CORES + lax.axis_index("c")
    gbuf = (gbuf0, gbuf1)
    tbuf = (tbuf0, tbuf1)
    gsem = (gsem0, gsem1)
    osem = (osem0, osem1)

    pltpu.async_copy(idxt_hbm.at[:, bt], idx_v, isem).wait()

    viota = lax.broadcasted_iota(jnp.int32, (16,), 0)
    esv = viota % 8
    etv0 = viota // 8
    etv1 = etv0 + 2

    def fire(g, b):
        for r in range(TG):
            pltpu.async_copy(
                table_hbm.at[idx_v.at[g * TG + r]],
                gbuf[b].at[r],
                gsem[b],
            )

    def drain_gathers(b):
        for r in range(TG):
            pltpu.make_async_copy(
                table_hbm.at[pl.ds(0, BPW)], gbuf[b].at[r], gsem[b]
            ).wait()

    def transpose(b):
        gb, tb = gbuf[b], tbuf[b]
        for tq in range(TG):
            tv = jnp.full((16,), tq, jnp.int32)

            @plsc.parallel_loop(0, BPW, unroll=4)
            def _tr_body(bl):
                blv = jnp.full((16,), 0, jnp.int32) + bl
                v0 = gb[tq, bl, pl.ds(0, 16)]
                v1 = gb[tq, bl, pl.ds(16, 16)]
                plsc.store_scatter(tb, [tv, etv0, esv, blv], v0)
                plsc.store_scatter(tb, [tv, etv1, esv, blv], v1)

    def out_start(g, b):
        pltpu.async_copy(
            tbuf[b].at[:, :, :, pl.ds(0, BPW)],
            out_hbm.at[pl.ds(g * TG, TG), :, bt],
            osem[b],
        )

    def out_wait(b):
        pltpu.make_async_copy(
            tbuf[b].at[:, :, :, pl.ds(0, BPW)],
            out_hbm.at[pl.ds(0, TG), :, bt],
            osem[b],
        ).wait()

    fire(0, 0)

    def phase(g, b, first, last):
        drain_gathers(b)
        if not last:
            fire(g + 1, 1 - b)
        if not first:
            out_wait(b)
        transpose(b)
        out_start(g, b)

    phase(0, 0, first=True, last=False)
    phase(1, 1, first=True, last=False)

    def steady(i, carry):
        g = 2 * i
        phase(g, 0, first=False, last=False)
        phase(g + 1, 1, first=False, last=False)
        return carry

    lax.fori_loop(1, NGT // 2 - 1, steady, 0)

    phase(NGT - 2, 0, first=False, last=False)
    phase(NGT - 1, 1, first=False, last=True)
    out_wait(0)
    out_wait(1)


_emb = pl.kernel(
    _emb_body,
    out_type=jax.ShapeDtypeStruct((SEQ, 4, NUM_WORKERS, 8, BPW), jnp.float32),
    mesh=plsc.VectorSubcoreMesh(core_axis_name="c", subcore_axis_name="s"),
    scratch_types=[
        pltpu.VMEM((SEQ, BPW), jnp.int32),
        pltpu.VMEM((TG, BPW, EMBED_DIM), jnp.float32),
        pltpu.VMEM((TG, BPW, EMBED_DIM), jnp.float32),
        pltpu.VMEM((TG, 4, 8, LANE_PAD), jnp.float32),
        pltpu.VMEM((TG, 4, 8, LANE_PAD), jnp.float32),
        pltpu.SemaphoreType.DMA,
        pltpu.SemaphoreType.DMA,
        pltpu.SemaphoreType.DMA,
        pltpu.SemaphoreType.DMA,
        pltpu.SemaphoreType.DMA,
    ],
    compiler_params=pltpu.CompilerParams(
        use_tc_tiling_on_sc=False, needs_layout_passes=False),
)


@jax.jit
def kernel(x, table):
    xt = jnp.transpose(x.astype(jnp.int32)).reshape(SEQ, NUM_WORKERS, BPW)
    out5 = _emb(table, xt)
    return jnp.transpose(out5, (2, 4, 0, 1, 3)).reshape(NROWS, SEQ, EMBED_DIM)
